# baseline (speedup 1.0000x reference)
import functools

import jax
import jax.numpy as jnp
from jax import lax
from jax.experimental import pallas as pl
from jax.experimental.pallas import tpu as pltpu
from jax.experimental.pallas import tpu_sc as plsc

N_ROWS = 16384
DIM = 2048
NC = 2
NS = 16
NW = NC * NS
L = 16

ROWS_PER_W = N_ROWS // NW
BLK = 8
N_BLKS = ROWS_PER_W // BLK
GRPS = DIM // L
NIN = 2
NOUT = 4

_mesh = plsc.VectorSubcoreMesh(
    core_axis_name="c", subcore_axis_name="s", num_cores=NC, num_subcores=NS
)


@functools.partial(
    pl.kernel,
    out_type=jax.ShapeDtypeStruct((N_ROWS, DIM), jnp.float32),
    mesh=_mesh,
    scratch_types=[
        pltpu.VMEM((DIM + L,), jnp.int32),
        pltpu.VMEM((DIM,), jnp.int32),
        pltpu.VMEM((NIN, BLK, DIM), jnp.float32),
        pltpu.VMEM((NOUT, BLK, DIM), jnp.float32),
        pltpu.SemaphoreType.DMA((NIN,)),
        pltpu.SemaphoreType.DMA((NOUT,)),
    ],
    compiler_params=pltpu.CompilerParams(needs_layout_passes=False),
)
def _reorder_sc(x_hbm, perm_hbm, y_hbm, inv_v, perm_v, in_v, out_v, sin, sout):
    wid = lax.axis_index("s") * NC + lax.axis_index("c")
    base0 = wid * ROWS_PER_W

    pltpu.sync_copy(perm_hbm, perm_v)

    rvecs = [jnp.full((L,), r, jnp.int32) for r in range(BLK)]
    iota16 = lax.iota(jnp.int32, L)

    def binv(j, _):
        off = pl.multiple_of(j * L, L)
        idxj = perm_v[pl.ds(off, L)]
        plsc.store_scatter(inv_v, [idxj], iota16 + off)
        return ()

    lax.fori_loop(0, GRPS, binv, ())

    def in_copy(b, q):
        return pltpu.make_async_copy(
            x_hbm.at[pl.ds(base0 + b * BLK, BLK)], in_v.at[q], sin.at[q]
        )

    def out_copy(b, q):
        return pltpu.make_async_copy(
            out_v.at[q], y_hbm.at[pl.ds(base0 + b * BLK, BLK)], sout.at[q]
        )

    def compute(qi, qo):
        src = in_v.at[qi]
        dst = out_v.at[qo]

        def do_grp(j, inv_cur):
            off_next = pl.multiple_of(j * L + L, L)
            inv_next = inv_v[pl.ds(off_next, L)]
            off = pl.multiple_of(j * L, L)
            vals = [src[r, pl.ds(off, L)] for r in range(BLK)]
            for r in range(BLK):
                plsc.store_scatter(dst, [rvecs[r], inv_cur], vals[r])
            return inv_next

        inv0 = inv_v[pl.ds(0, L)]
        lax.fori_loop(0, GRPS, do_grp, inv0, unroll=2)

    in_copy(0, 0).start()
    in_copy(1, 1).start()

    def quad_body(p, _):
        for k in range(4):
            b = 4 * p + k
            qi = k % NIN
            qo = k % NOUT
            in_copy(b, qi).wait()

            @pl.when(b >= NOUT)
            def _wait_out():
                out_copy(b - NOUT, qo).wait()

            compute(qi, qo)
            out_copy(b, qo).start()

            @pl.when(b + 2 < N_BLKS)
            def _next_in():
                in_copy(b + 2, qi).start()

        return ()

    lax.fori_loop(0, N_BLKS // 4, quad_body, ())
    out_copy(N_BLKS - 4, 0).wait()
    out_copy(N_BLKS - 3, 1).wait()
    out_copy(N_BLKS - 2, 2).wait()
    out_copy(N_BLKS - 1, 3).wait()


def kernel(x, logp, randperm):
    y = _reorder_sc(x, randperm)
    if logp is None:
        return y
    return (y, logp)


# speedup vs baseline: 1.0346x; 1.0346x over previous
import functools

import jax
import jax.numpy as jnp
from jax import lax
from jax.experimental import pallas as pl
from jax.experimental.pallas import tpu as pltpu
from jax.experimental.pallas import tpu_sc as plsc

N_ROWS = 16384
DIM = 2048
NC = 2
NS = 16
NW = NC * NS
L = 16

ROWS_PER_W = N_ROWS // NW
BLK = 8
N_BLKS = ROWS_PER_W // BLK
GRPS = DIM // L
NIN = 4
NOUT = 2

_mesh = plsc.VectorSubcoreMesh(
    core_axis_name="c", subcore_axis_name="s", num_cores=NC, num_subcores=NS
)


@functools.partial(
    pl.kernel,
    out_type=jax.ShapeDtypeStruct((N_ROWS, DIM), jnp.float32),
    mesh=_mesh,
    scratch_types=[
        pltpu.VMEM((DIM + L,), jnp.int32),
        pltpu.VMEM((DIM,), jnp.int32),
        pltpu.VMEM((NIN, BLK, DIM), jnp.float32),
        pltpu.VMEM((NOUT, BLK, DIM), jnp.float32),
        pltpu.SemaphoreType.DMA((NIN,)),
        pltpu.SemaphoreType.DMA((NOUT,)),
    ],
    compiler_params=pltpu.CompilerParams(needs_layout_passes=False),
)
def _reorder_sc(x_hbm, perm_hbm, y_hbm, inv_v, perm_v, in_v, out_v, sin, sout):
    wid = lax.axis_index("s") * NC + lax.axis_index("c")
    base0 = wid * ROWS_PER_W

    pltpu.sync_copy(perm_hbm, perm_v)

    rvecs = [jnp.full((L,), r, jnp.int32) for r in range(BLK)]
    iota16 = lax.iota(jnp.int32, L)

    def binv(j, _):
        off = pl.multiple_of(j * L, L)
        idxj = perm_v[pl.ds(off, L)]
        plsc.store_scatter(inv_v, [idxj], iota16 + off)
        return ()

    lax.fori_loop(0, GRPS, binv, ())

    def in_copy(b, q):
        return pltpu.make_async_copy(
            x_hbm.at[pl.ds(base0 + b * BLK, BLK)], in_v.at[q], sin.at[q]
        )

    def out_copy(b, q):
        return pltpu.make_async_copy(
            out_v.at[q], y_hbm.at[pl.ds(base0 + b * BLK, BLK)], sout.at[q]
        )

    def compute(qi, qo):
        src = in_v.at[qi]
        dst = out_v.at[qo]

        def do_grp(j, inv_cur):
            off_next = pl.multiple_of(j * L + L, L)
            inv_next = inv_v[pl.ds(off_next, L)]
            off = pl.multiple_of(j * L, L)
            vals = [src[r, pl.ds(off, L)] for r in range(BLK)]
            for r in range(BLK):
                plsc.store_scatter(dst, [rvecs[r], inv_cur], vals[r])
            return inv_next

        inv0 = inv_v[pl.ds(0, L)]
        lax.fori_loop(0, GRPS, do_grp, inv0, unroll=2)

    in_copy(0, 0).start()
    in_copy(1, 1).start()
    in_copy(2, 2).start()

    def quad_body(p, _):
        for k in range(NIN):
            b = 4 * p + k
            qo = b % NOUT
            in_copy(b, k).wait()

            @pl.when(b + 3 < N_BLKS)
            def _next_in():
                in_copy(b + 3, (k + 3) % NIN).start()

            @pl.when(b >= NOUT)
            def _wait_out():
                out_copy(b - NOUT, qo).wait()

            compute(k, qo)
            out_copy(b, qo).start()

        return ()

    lax.fori_loop(0, N_BLKS // NIN, quad_body, ())
    out_copy(N_BLKS - 2, 0).wait()
    out_copy(N_BLKS - 1, 1).wait()


def kernel(x, logp, randperm):
    y = _reorder_sc(x, randperm)
    if logp is None:
        return y
    return (y, logp)
